# baseline (speedup 1.0000x reference)
import functools

import jax
import jax.numpy as jnp
from jax import lax
from jax.experimental import pallas as pl
from jax.experimental.pallas import tpu as pltpu
from jax.experimental.pallas import tpu_sc as plsc

_MAX_RANGE = 1000
_BATCH = 16384
_NUM_CORES = 2
_NUM_SUBCORES = 16
_NW = _NUM_CORES * _NUM_SUBCORES
_CHUNK = _BATCH // _NW
_NBUCKET = 8
_BWIDTH = 128
_GROUP = 64
_NGROUP = _CHUNK // _GROUP
_MAGIC = 8388608.0


@functools.partial(
    pl.kernel,
    mesh=plsc.VectorSubcoreMesh(core_axis_name="c", subcore_axis_name="s"),
    out_type=jax.ShapeDtypeStruct((_BATCH,), jnp.float32),
    scratch_types=[
        pltpu.VMEM((_CHUNK,), jnp.float32),
        pltpu.VMEM((_NBUCKET * _CHUNK,), jnp.int32),
        pltpu.VMEM((_CHUNK,), jnp.int32),
        pltpu.VMEM((_CHUNK,), jnp.int32),
        pltpu.VMEM((_NBUCKET * _GROUP, 1, _BWIDTH), jnp.float32),
        pltpu.VMEM((_CHUNK,), jnp.float32),
        pltpu.SemaphoreType.DMA,
    ],
)
def _gather(x_hbm, param_hbm, out_hbm, x_v, idx_v, o_v, b_v, bufs, out_v, sem):
    wid = lax.axis_index("s") * _NUM_CORES + lax.axis_index("c")
    base = wid * _CHUNK
    pltpu.sync_copy(x_hbm.at[pl.ds(base, _CHUNK)], x_v)
    lane = lax.iota(jnp.int32, 16)
    neg1 = jnp.full((16,), -1, jnp.int32)
    for t in range(_CHUNK // 16):
        off = t * 16
        xv = x_v[pl.ds(off, 16)]
        xs = xv * float(_MAX_RANGE - 1)
        rounded = (xs + _MAGIC) - _MAGIC
        col = rounded.astype(jnp.int32)
        bkt = lax.shift_right_logical(col, 7)
        o_v[pl.ds(off, 16)] = col & (_BWIDTH - 1)
        b_v[pl.ds(off, 16)] = bkt
        rows = base + off + lane
        for k in range(_NBUCKET):
            idx_v[pl.ds(k * _CHUNK + off, 16)] = jnp.where(bkt == k, rows, neg1)
    flat2d = bufs.reshape(_NBUCKET * _GROUP, _BWIDTH)
    for g in range(_NGROUP):
        copies = []
        for k in range(_NBUCKET):
            if k < _NBUCKET - 1:
                bstart_k = k * _BWIDTH
            else:
                bstart_k = pl.multiple_of(
                    jnp.int32((_NBUCKET - 1) * _BWIDTH) + wid * 0, _BWIDTH)
            idx_ref = idx_v.at[pl.ds(k * _CHUNK + g * _GROUP, _GROUP)]
            src = param_hbm.at[
                plsc.Indices(idx_ref, ignored_value=-1), pl.ds(0, 1),
                pl.ds(bstart_k, _BWIDTH)]
            dst = bufs.at[pl.ds(k * _GROUP, _GROUP), pl.ds(0, 1),
                          pl.ds(0, _BWIDTH)]
            copies.append(pltpu.async_copy(src, dst, sem))
        for cp in copies:
            cp.wait()
        for t in range(_GROUP // 16):
            off = g * _GROUP + t * 16
            o_vec = o_v[pl.ds(off, 16)]
            b_vec = b_v[pl.ds(off, 16)]
            acc = jnp.zeros((16,), jnp.float32)
            for j in range(16):
                slot = b_vec[j] * _GROUP + t * 16 + j
                v = flat2d[slot, pl.ds(o_vec[j], 16)]
                acc = jnp.where(lane == j, v[0], acc)
            out_v[pl.ds(off, 16)] = acc
    pltpu.sync_copy(out_v, out_hbm.at[pl.ds(base, _CHUNK)])


def kernel(x, param):
    return _gather(x, param).reshape(_BATCH, 1, 1)


# speedup vs baseline: 5.5795x; 5.5795x over previous
import functools

import jax
import jax.numpy as jnp
from jax import lax
from jax.experimental import pallas as pl
from jax.experimental.pallas import tpu as pltpu
from jax.experimental.pallas import tpu_sc as plsc

_MAX_RANGE = 1000
_BATCH = 16384
_COLS = 128
_NUM_CORES = 2
_NUM_SUBCORES = 16
_NW = _NUM_CORES * _NUM_SUBCORES
_ROWS_PER_W = (_BATCH // _COLS) // _NW
_MAGIC = 8388608.0


@functools.partial(
    pl.kernel,
    mesh=plsc.VectorSubcoreMesh(core_axis_name="c", subcore_axis_name="s"),
    out_type=jax.ShapeDtypeStruct((_BATCH // _COLS, _COLS), jnp.float32),
    scratch_types=[
        pltpu.VMEM((_ROWS_PER_W, _COLS), jnp.float32),
        pltpu.VMEM((_ROWS_PER_W, _COLS), jnp.int32),
        pltpu.VMEM((_ROWS_PER_W, _COLS), jnp.float32),
        pltpu.SemaphoreType.DMA,
    ],
)
def _gather(x_hbm, param_hbm, out_hbm, x_v, idx_v, gat_v, sem):
    wid = lax.axis_index("s") * _NUM_CORES + lax.axis_index("c")
    row0 = wid * _ROWS_PER_W
    pltpu.sync_copy(x_hbm.at[pl.ds(row0, _ROWS_PER_W)], x_v)
    lane = lax.iota(jnp.int32, 16)
    for j in range(_ROWS_PER_W):
        for c in range(_COLS // 16):
            xv = x_v[j, pl.ds(c * 16, 16)]
            xs = xv * float(_MAX_RANGE - 1)
            rounded = (xs + _MAGIC) - _MAGIC
            col = rounded.astype(jnp.int32)
            base = (row0 + j) * _COLS + c * 16
            idx_v[j, pl.ds(c * 16, 16)] = (base + lane) * _MAX_RANGE + col
    copies = [
        pltpu.async_copy(param_hbm.at[idx_v.at[j]], gat_v.at[j], sem)
        for j in range(_ROWS_PER_W)
    ]
    for cp in copies:
        cp.wait()
    pltpu.sync_copy(gat_v, out_hbm.at[pl.ds(row0, _ROWS_PER_W)])


def kernel(x, param):
    x2 = x.reshape(_BATCH // _COLS, _COLS)
    pflat = param.reshape(_BATCH * _MAX_RANGE)
    out = _gather(x2, pflat)
    return out.reshape(_BATCH, 1, 1)
